# Initial kernel scaffold; baseline (speedup 1.0000x reference)
import jax
import jax.numpy as jnp
from jax.experimental import pallas as pl


def kernel(labels, train, table1, table2, W1, b1, W2, b2):
    raise NotImplementedError("write your pallas kernel here")


# speedup vs baseline: 1.5619x; 1.5619x over previous
import functools

import jax
import jax.numpy as jnp
from jax import lax
from jax.experimental import pallas as pl
from jax.experimental.pallas import tpu as pltpu
from jax.experimental.pallas import tpu_sc as plsc

NUM_CLASSES = 100000
DIM = 128
BATCH = 16384

NC = 2
NS = 16
NW = NC * NS
B_PER_W = BATCH // NW
CHUNK = 128
N_CHUNKS = B_PER_W // CHUNK


def _sc_gather_body(labels_hbm, t1_hbm, t2_hbm, out_hbm, idx_v, buf1, buf2,
                    sem1, sem2):
    wid = lax.axis_index("s") * NC + lax.axis_index("c")
    pltpu.sync_copy(labels_hbm.at[pl.ds(wid * N_CHUNKS, N_CHUNKS)], idx_v)
    for c in range(N_CHUNKS):
        idx_c = idx_v.at[c]
        cp1 = pltpu.async_copy(t1_hbm.at[idx_c], buf1, sem1)
        cp2 = pltpu.async_copy(t2_hbm.at[idx_c], buf2, sem2)
        cp1.wait()
        cp2.wait()
        row0 = (wid * N_CHUNKS + c) * CHUNK
        pltpu.sync_copy(buf1, out_hbm.at[pl.ds(row0, CHUNK), 0])
        pltpu.sync_copy(buf2, out_hbm.at[pl.ds(row0, CHUNK), 1])


@functools.partial(jax.jit, static_argnames=())
def _sc_gather(labels2d, table1, table2):
    mesh = plsc.VectorSubcoreMesh(
        core_axis_name="c", subcore_axis_name="s",
        num_cores=NC, num_subcores=NS)
    k = pl.kernel(
        _sc_gather_body,
        out_type=jax.ShapeDtypeStruct((BATCH, 2, DIM), jnp.float32),
        mesh=mesh,
        scratch_types=[
            pltpu.VMEM((N_CHUNKS, CHUNK), jnp.int32),
            pltpu.VMEM((CHUNK, DIM), jnp.float32),
            pltpu.VMEM((CHUNK, DIM), jnp.float32),
            pltpu.SemaphoreType.DMA,
            pltpu.SemaphoreType.DMA,
        ],
    )
    return k(labels2d, table1, table2)


def _mlp_body(emb_ref, w1_ref, b1_ref, w2_ref, b2_ref, out_ref):
    e1 = emb_ref[:, 0, :]
    e2 = emb_ref[:, 1, :]
    w1a = w1_ref[:DIM, :]
    w1b = w1_ref[DIM:, :]
    h = (jnp.dot(e1, w1a, preferred_element_type=jnp.float32)
         + jnp.dot(e2, w1b, preferred_element_type=jnp.float32)
         + b1_ref[0, :][None, :])
    h = h * jax.nn.sigmoid(h)
    g = jnp.dot(h, w2_ref[...], preferred_element_type=jnp.float32)
    out_ref[...] = g + b2_ref[0, :][None, :]


def _mlp(emb, W1, b1, W2, b2):
    bb = 2048
    grid = (BATCH // bb,)
    return pl.pallas_call(
        _mlp_body,
        grid=grid,
        in_specs=[
            pl.BlockSpec((bb, 2, DIM), lambda i: (i, 0, 0)),
            pl.BlockSpec((2 * DIM, DIM), lambda i: (0, 0)),
            pl.BlockSpec((1, DIM), lambda i: (0, 0)),
            pl.BlockSpec((DIM, DIM), lambda i: (0, 0)),
            pl.BlockSpec((1, DIM), lambda i: (0, 0)),
        ],
        out_specs=pl.BlockSpec((bb, DIM), lambda i: (i, 0)),
        out_shape=jax.ShapeDtypeStruct((BATCH, DIM), jnp.float32),
    )(emb, W1, b1, W2, b2)


def kernel(labels, train, table1, table2, W1, b1, W2, b2):
    labels2d = labels.astype(jnp.int32).reshape(BATCH // CHUNK, CHUNK)
    embeddings = _sc_gather(labels2d, table1, table2)
    global_embeddings = _mlp(embeddings, W1, b1.reshape(1, DIM),
                             W2, b2.reshape(1, DIM))
    return (embeddings, global_embeddings)
